# baseline (speedup 1.0000x reference)
import functools

import jax
import jax.numpy as jnp
from jax.experimental import pallas as pl
from jax.experimental.pallas import tpu as pltpu


def _relu(v):
    return jnp.maximum(v, 0.0)


def _graph_ln(h, w, b):
    m = h.mean()
    hc = h - m
    s = jnp.sqrt(jnp.mean(hc * hc) + 1e-5)
    return hc / s * w + b


def _gcn(h, src, dst, N, w, b):
    hw = h @ w
    deg = jnp.zeros((N,), h.dtype).at[dst].add(1.0)
    dinv = jax.lax.rsqrt(jnp.maximum(deg, 1.0))
    coef = dinv[src] * dinv[dst]
    out = jnp.zeros_like(hw).at[dst].add(hw[src] * coef[:, None])
    return out + b


def _gat(h, src, dst, N, w, asrc, adst, b):
    hw = h @ w
    a_s = hw @ asrc
    a_d = hw @ adst
    e = a_s[src] + a_d[dst]
    e = jnp.where(e > 0, e, 0.2 * e)
    emax = jax.ops.segment_max(e, dst, num_segments=N)
    emax = jnp.where(jnp.isfinite(emax), emax, 0.0)
    ee = jnp.exp(e - emax[dst])
    den = jax.ops.segment_sum(ee, dst, num_segments=N)
    coef = ee / (den[dst] + 1e-16)
    out = jnp.zeros_like(hw).at[dst].add(hw[src] * coef[:, None])
    return out + b


def _dual(h, src, dst, N, p, br, l, wmix):
    g = _relu(_graph_ln(_gcn(h, src, dst, N, p['gcn_%s%d_w' % (br, l)], p['gcn_%s%d_b' % (br, l)]),
                        p['ln_gcn_%s%d_w' % (br, l)], p['ln_gcn_%s%d_b' % (br, l)]))
    a = _relu(_graph_ln(_gat(h, src, dst, N, p['gat_%s%d_w' % (br, l)], p['gat_%s%d_asrc' % (br, l)],
                             p['gat_%s%d_adst' % (br, l)], p['gat_%s%d_b' % (br, l)]),
                        p['ln_gat_%s%d_w' % (br, l)], p['ln_gat_%s%d_b' % (br, l)]))
    return wmix * g + (1.0 - wmix) * a


def _roi_align(feat, boxes, out_size=7):
    f = feat[0]
    C, H, W = f.shape
    bx1, by1, bx2, by2 = boxes[:, 1], boxes[:, 2], boxes[:, 3], boxes[:, 4]
    rw = jnp.maximum(bx2 - bx1, 1.0)
    rh = jnp.maximum(by2 - by1, 1.0)
    off = (jnp.arange(out_size, dtype=f.dtype) + 0.5) / out_size
    ys = jnp.clip(by1[:, None] + off[None, :] * rh[:, None], 0.0, H - 1.0)
    xs = jnp.clip(bx1[:, None] + off[None, :] * rw[:, None], 0.0, W - 1.0)
    y0 = jnp.floor(ys); x0 = jnp.floor(xs)
    y0i = y0.astype(jnp.int32); x0i = x0.astype(jnp.int32)
    y1i = jnp.minimum(y0i + 1, H - 1); x1i = jnp.minimum(x0i + 1, W - 1)
    ly = ys - y0; lx = xs - x0
    K = boxes.shape[0]
    Y0 = jnp.broadcast_to(y0i[:, :, None], (K, out_size, out_size))
    Y1 = jnp.broadcast_to(y1i[:, :, None], (K, out_size, out_size))
    X0 = jnp.broadcast_to(x0i[:, None, :], (K, out_size, out_size))
    X1 = jnp.broadcast_to(x1i[:, None, :], (K, out_size, out_size))
    LY = jnp.broadcast_to(ly[:, :, None], (K, out_size, out_size))
    LX = jnp.broadcast_to(lx[:, None, :], (K, out_size, out_size))
    v00 = f[:, Y0, X0]; v01 = f[:, Y0, X1]; v10 = f[:, Y1, X0]; v11 = f[:, Y1, X1]
    out = v00 * (1 - LY) * (1 - LX) + v01 * (1 - LY) * LX + v10 * LY * (1 - LX) + v11 * LY * LX
    return jnp.transpose(out, (1, 0, 2, 3))


def _mlp2_body(x_ref, w1_ref, b1_ref, w2_ref, b2_ref, o_ref):
    h = jnp.maximum(
        jnp.dot(x_ref[...], w1_ref[...], preferred_element_type=jnp.float32)
        + b1_ref[...], 0.0)
    o_ref[...] = jnp.maximum(
        jnp.dot(h, w2_ref[...], preferred_element_type=jnp.float32)
        + b2_ref[...], 0.0)


def _mlp2_pallas(x, w1, b1, w2, b2, tile=2048):
    M, K = x.shape
    H = w1.shape[1]
    N = w2.shape[1]
    Mp = ((M + tile - 1) // tile) * tile
    if Mp != M:
        x = jnp.pad(x, ((0, Mp - M), (0, 0)))
    out = pl.pallas_call(
        _mlp2_body,
        grid=(Mp // tile,),
        in_specs=[
            pl.BlockSpec((tile, K), lambda i: (i, 0)),
            pl.BlockSpec((K, H), lambda i: (0, 0)),
            pl.BlockSpec((1, H), lambda i: (0, 0)),
            pl.BlockSpec((H, N), lambda i: (0, 0)),
            pl.BlockSpec((1, N), lambda i: (0, 0)),
        ],
        out_specs=pl.BlockSpec((tile, N), lambda i: (i, 0)),
        out_shape=jax.ShapeDtypeStruct((Mp, N), jnp.float32),
    )(x, w1, b1.reshape(1, H), w2, b2.reshape(1, N))
    return out[:M]


def kernel(x, img, imgpos, text, params, edge_index, edge_pairing_index):
    p = params
    N = x.shape[0]
    ROI = 7
    sl = jnp.arange(N, dtype=edge_index.dtype)
    src = jnp.concatenate([edge_index[0], sl])
    dst = jnp.concatenate([edge_index[1], sl])

    def mlp2(h, n1, n2):
        h = _relu(h @ p[n1 + '_w'] + p[n1 + '_b'])
        return _relu(h @ p[n2 + '_w'] + p[n2 + '_b'])

    box = mlp2(x, 'pre1', 'pre2')
    box = _dual(box, src, dst, N, p, 'box', 1, p['w_bbox'][0])
    box = _dual(box, src, dst, N, p, 'box', 2, p['w_bbox'][1])
    box_row = mlp2(box, 'post_box_row1', 'post_box_row2')
    box_col = mlp2(box, 'post_box_col1', 'post_box_col2')

    dec = jax.lax.conv_general_dilated(img, p['conv_w'], (1, 1), 'SAME',
                                       dimension_numbers=('NCHW', 'OIHW', 'NCHW'))
    dec = _relu(dec + p['conv_b'][None, :, None, None])
    roi = _roi_align(dec, imgpos, ROI)
    h = roi.reshape(roi.shape[0], -1)
    h = _relu(h @ p['emb1_w'] + p['emb1_b'])
    h = _relu(h @ p['emb2_w'] + p['emb2_b'])
    imgf = _relu(h @ p['emb3_w'] + p['emb3_b'])
    imgf = _dual(imgf, src, dst, N, p, 'img', 1, p['w_img'][0])
    imgf = _dual(imgf, src, dst, N, p, 'img', 2, p['w_img'][1])
    img_row = mlp2(imgf, 'post_img_row1', 'post_img_row2')
    img_col = mlp2(imgf, 'post_img_col1', 'post_img_col2')

    tf = mlp2(text, 'pret1', 'pret2')
    tf = _dual(tf, src, dst, N, p, 'text', 1, p['w_text'][0])
    tf = _dual(tf, src, dst, N, p, 'text', 2, p['w_text'][1])
    text_row = mlp2(tf, 'post_text_row1', 'post_text_row2')
    text_col = mlp2(tf, 'post_text_col1', 'post_text_col2')

    fr = mlp2(jnp.concatenate([box_row, img_row, text_row], axis=1), 'fus_row1', 'fus_row2')
    fc = mlp2(jnp.concatenate([box_col, img_col, text_col], axis=1), 'fus_col1', 'fus_col2')

    s, o = edge_pairing_index[0], edge_pairing_index[1]
    row_feat = jnp.concatenate([fr[s], fr[o]], axis=1)
    col_feat = jnp.concatenate([fc[s], fc[o]], axis=1)
    cls_row = _mlp2_pallas(row_feat, p['cls_row1_w'], p['cls_row1_b'],
                           p['cls_row2_w'], p['cls_row2_b'])
    cls_col = _mlp2_pallas(col_feat, p['cls_col1_w'], p['cls_col1_b'],
                           p['cls_col2_w'], p['cls_col2_b'])
    return (cls_row, cls_col)


# speedup vs baseline: 1.3450x; 1.3450x over previous
import functools

import jax
import jax.numpy as jnp
from jax import lax
from jax.experimental import pallas as pl
from jax.experimental.pallas import tpu as pltpu
from jax.experimental.pallas import tpu_sc as plsc

_N = 10000
_NPAD = 10240
_D = 512
_F = 4
_K = 128
_G = 2
_GK = _G * _K
_NS = 16
_E = 330000
_CH = -(-_E // (_NS * _GK))
_EPAD = _NS * _GK * _CH
_ZROWS = _NPAD // _NS
_OROWS = _N // _NS


def _agg_body_p(metaI, table, zeros, out, acc_sh, meta_v, rows_v, sem):
    core = lax.axis_index("c")
    sub = lax.axis_index("s")
    for fp in range(2):
        f = core * 2 + fp
        pltpu.sync_copy(zeros.at[pl.ds(sub * _ZROWS, _ZROWS)],
                        acc_sh.at[pl.ds(sub * _ZROWS, _ZROWS)])
        plsc.subcore_barrier()

        def body(c, carry):
            lin = (f * _NS + sub) * _CH + c
            pltpu.sync_copy(metaI.at[lin], meta_v)
            cps = [pltpu.async_copy(table.at[meta_v.at[0, g]],
                                    rows_v.at[pl.ds(g * _K, _K)], sem)
                   for g in range(_G)]
            for cp in cps:
                cp.wait()
            for g in range(_G):
                pltpu.sync_copy(rows_v.at[pl.ds(g * _K, _K)],
                                acc_sh.at[meta_v.at[1, g]], add=True)
            return carry

        lax.fori_loop(0, _CH, body, 0)
        plsc.subcore_barrier()
        pltpu.sync_copy(acc_sh.at[pl.ds(sub * _ZROWS, _ZROWS)],
                        out.at[pl.ds(f * _NPAD + sub * _ZROWS, _ZROWS)])
        plsc.subcore_barrier()


def _agg_body_w(metaI, metaW, table, zeros, out, acc_sh, meta_v, wgt_v, rows_v, sem):
    core = lax.axis_index("c")
    sub = lax.axis_index("s")
    for fp in range(2):
        f = core * 2 + fp
        pltpu.sync_copy(zeros.at[pl.ds(sub * _ZROWS, _ZROWS)],
                        acc_sh.at[pl.ds(sub * _ZROWS, _ZROWS)])
        plsc.subcore_barrier()

        def body(c, carry):
            lin = (f * _NS + sub) * _CH + c
            pltpu.sync_copy(metaI.at[lin], meta_v)
            pltpu.sync_copy(metaW.at[sub * _CH + c], wgt_v)
            cps = [pltpu.async_copy(table.at[meta_v.at[0, g]],
                                    rows_v.at[pl.ds(g * _K, _K)], sem)
                   for g in range(_G)]
            for cp in cps:
                cp.wait()

            def wbody(row, _):
                wb = wgt_v[row // 8, pl.ds((row % 8) * 16, 16)]
                for cc in range(_D // _F // 16):
                    rows_v[row, pl.ds(cc * 16, 16)] = (
                        rows_v[row, pl.ds(cc * 16, 16)] * wb)
                return 0

            lax.fori_loop(0, _GK, wbody, 0)
            for g in range(_G):
                pltpu.sync_copy(rows_v.at[pl.ds(g * _K, _K)],
                                acc_sh.at[meta_v.at[1, g]], add=True)
            return carry

        lax.fori_loop(0, _CH, body, 0)
        plsc.subcore_barrier()
        pltpu.sync_copy(acc_sh.at[pl.ds(sub * _ZROWS, _ZROWS)],
                        out.at[pl.ds(f * _NPAD + sub * _ZROWS, _ZROWS)])
        plsc.subcore_barrier()


_MESH = plsc.VectorSubcoreMesh(core_axis_name="c", subcore_axis_name="s")
_OUT_T = jax.ShapeDtypeStruct((_F * _NPAD, 128), jnp.float32)

_AGG_P = pl.kernel(
    _agg_body_p, out_type=_OUT_T, mesh=_MESH,
    scratch_types=[
        pltpu.VMEM_SHARED((_NPAD, 128), jnp.float32),
        pltpu.VMEM((2, _G, _K), jnp.int32),
        pltpu.VMEM((_GK, 128), jnp.float32),
        pltpu.SemaphoreType.DMA,
    ])

_AGG_W = pl.kernel(
    _agg_body_w, out_type=_OUT_T, mesh=_MESH,
    scratch_types=[
        pltpu.VMEM_SHARED((_NPAD, 128), jnp.float32),
        pltpu.VMEM((2, _G, _K), jnp.int32),
        pltpu.VMEM((_GK // 8, 128), jnp.float32),
        pltpu.VMEM((_GK, 128), jnp.float32),
        pltpu.SemaphoreType.DMA,
    ])


def _prep_edges(src, dst):
    pad = _EPAD - _E
    srcp = jnp.concatenate([src, jnp.zeros((pad,), jnp.int32)])
    dstp = jnp.concatenate([dst, jnp.full((pad,), _N, jnp.int32)])
    src4 = srcp[None, :] + (jnp.arange(_F, dtype=jnp.int32) * _NPAD)[:, None]
    sI = src4.reshape(_F, _NS, _CH, 1, _G, _K)
    dI = jnp.broadcast_to(dstp.reshape(1, _NS, _CH, 1, _G, _K),
                          (_F, _NS, _CH, 1, _G, _K))
    return jnp.concatenate([sI, dI], axis=3).reshape(_F * _NS * _CH, 2, _G, _K)


def _prep_wgt(w):
    wp = jnp.concatenate([w, jnp.zeros((_EPAD - _E,), jnp.float32)])
    wp = jnp.broadcast_to(wp.reshape(_NS * _CH, _GK // 8, 8, 1),
                          (_NS * _CH, _GK // 8, 8, 16))
    return wp.reshape(_NS * _CH, _GK // 8, 128)


def _agg_run(table, metaI, metaW=None):
    t4 = table.reshape(_N, _F, 128).transpose(1, 0, 2)
    t4 = jnp.pad(t4, ((0, 0), (0, _NPAD - _N), (0, 0))).reshape(_F * _NPAD, 128)
    zeros = jnp.zeros((_NPAD, 128), jnp.float32)
    if metaW is None:
        o = _AGG_P(metaI, t4, zeros)
    else:
        o = _AGG_W(metaI, metaW, t4, zeros)
    return o.reshape(_F, _NPAD, 128)[:, :_N].transpose(1, 0, 2).reshape(_N, _D)


def _relu(v):
    return jnp.maximum(v, 0.0)


def _graph_ln(h, w, b):
    m = h.mean()
    hc = h - m
    s = jnp.sqrt(jnp.mean(hc * hc) + 1e-5)
    return hc / s * w + b


def _gcn_sc(h, src, dst, metaI, dinv, w, b):
    hw = h @ w
    coef = dinv[src] * dinv[dst]
    return _agg_run(hw, metaI, _prep_wgt(coef)) + b


def _gat_sc(h, src, dst, metaI, w, asrc, adst, b):
    hw = h @ w
    a_s = hw @ asrc
    a_d = hw @ adst
    e = a_s[src] + a_d[dst]
    e = jnp.where(e > 0, e, 0.2 * e)
    emax = jax.ops.segment_max(e, dst, num_segments=_N)
    emax = jnp.where(jnp.isfinite(emax), emax, 0.0)
    ee = jnp.exp(e - emax[dst])
    den = jax.ops.segment_sum(ee, dst, num_segments=_N)
    coef = ee / (den[dst] + 1e-16)
    return _agg_run(hw, metaI, _prep_wgt(coef)) + b


def _dual_sc(h, src, dst, metaI, dinv, p, br, l, wmix):
    g = _relu(_graph_ln(
        _gcn_sc(h, src, dst, metaI, dinv, p['gcn_%s%d_w' % (br, l)], p['gcn_%s%d_b' % (br, l)]),
        p['ln_gcn_%s%d_w' % (br, l)], p['ln_gcn_%s%d_b' % (br, l)]))
    a = _relu(_graph_ln(
        _gat_sc(h, src, dst, metaI, p['gat_%s%d_w' % (br, l)], p['gat_%s%d_asrc' % (br, l)],
                p['gat_%s%d_adst' % (br, l)], p['gat_%s%d_b' % (br, l)]),
        p['ln_gat_%s%d_w' % (br, l)], p['ln_gat_%s%d_b' % (br, l)]))
    return wmix * g + (1.0 - wmix) * a


def _roi_align(feat, boxes, out_size=7):
    f = feat[0]
    C, H, W = f.shape
    bx1, by1, bx2, by2 = boxes[:, 1], boxes[:, 2], boxes[:, 3], boxes[:, 4]
    rw = jnp.maximum(bx2 - bx1, 1.0)
    rh = jnp.maximum(by2 - by1, 1.0)
    off = (jnp.arange(out_size, dtype=f.dtype) + 0.5) / out_size
    ys = jnp.clip(by1[:, None] + off[None, :] * rh[:, None], 0.0, H - 1.0)
    xs = jnp.clip(bx1[:, None] + off[None, :] * rw[:, None], 0.0, W - 1.0)
    y0 = jnp.floor(ys); x0 = jnp.floor(xs)
    y0i = y0.astype(jnp.int32); x0i = x0.astype(jnp.int32)
    y1i = jnp.minimum(y0i + 1, H - 1); x1i = jnp.minimum(x0i + 1, W - 1)
    ly = ys - y0; lx = xs - x0
    K = boxes.shape[0]
    Y0 = jnp.broadcast_to(y0i[:, :, None], (K, out_size, out_size))
    Y1 = jnp.broadcast_to(y1i[:, :, None], (K, out_size, out_size))
    X0 = jnp.broadcast_to(x0i[:, None, :], (K, out_size, out_size))
    X1 = jnp.broadcast_to(x1i[:, None, :], (K, out_size, out_size))
    LY = jnp.broadcast_to(ly[:, :, None], (K, out_size, out_size))
    LX = jnp.broadcast_to(lx[:, None, :], (K, out_size, out_size))
    v00 = f[:, Y0, X0]; v01 = f[:, Y0, X1]; v10 = f[:, Y1, X0]; v11 = f[:, Y1, X1]
    out = v00 * (1 - LY) * (1 - LX) + v01 * (1 - LY) * LX + v10 * LY * (1 - LX) + v11 * LY * LX
    return jnp.transpose(out, (1, 0, 2, 3))


def _mlp2_body(x_ref, w1_ref, b1_ref, w2_ref, b2_ref, o_ref):
    h = jnp.maximum(
        jnp.dot(x_ref[...], w1_ref[...], preferred_element_type=jnp.float32)
        + b1_ref[...], 0.0)
    o_ref[...] = jnp.maximum(
        jnp.dot(h, w2_ref[...], preferred_element_type=jnp.float32)
        + b2_ref[...], 0.0)


def _mlp2_pallas(x, w1, b1, w2, b2, tile=2048):
    M, K = x.shape
    H = w1.shape[1]
    N = w2.shape[1]
    Mp = ((M + tile - 1) // tile) * tile
    if Mp != M:
        x = jnp.pad(x, ((0, Mp - M), (0, 0)))
    out = pl.pallas_call(
        _mlp2_body,
        grid=(Mp // tile,),
        in_specs=[
            pl.BlockSpec((tile, K), lambda i: (i, 0)),
            pl.BlockSpec((K, H), lambda i: (0, 0)),
            pl.BlockSpec((1, H), lambda i: (0, 0)),
            pl.BlockSpec((H, N), lambda i: (0, 0)),
            pl.BlockSpec((1, N), lambda i: (0, 0)),
        ],
        out_specs=pl.BlockSpec((tile, N), lambda i: (i, 0)),
        out_shape=jax.ShapeDtypeStruct((Mp, N), jnp.float32),
    )(x, w1, b1.reshape(1, H), w2, b2.reshape(1, N))
    return out[:M]


def kernel(x, img, imgpos, text, params, edge_index, edge_pairing_index):
    p = params
    N = x.shape[0]
    ROI = 7
    sl = jnp.arange(N, dtype=edge_index.dtype)
    src = jnp.concatenate([edge_index[0], sl])
    dst = jnp.concatenate([edge_index[1], sl])

    metaI = _prep_edges(src, dst)
    deg = jnp.zeros((N,), jnp.float32).at[dst].add(1.0)
    dinv = jax.lax.rsqrt(jnp.maximum(deg, 1.0))

    def mlp2(h, n1, n2):
        h = _relu(h @ p[n1 + '_w'] + p[n1 + '_b'])
        return _relu(h @ p[n2 + '_w'] + p[n2 + '_b'])

    box = mlp2(x, 'pre1', 'pre2')
    box = _dual_sc(box, src, dst, metaI, dinv, p, 'box', 1, p['w_bbox'][0])
    box = _dual_sc(box, src, dst, metaI, dinv, p, 'box', 2, p['w_bbox'][1])
    box_row = mlp2(box, 'post_box_row1', 'post_box_row2')
    box_col = mlp2(box, 'post_box_col1', 'post_box_col2')

    dec = jax.lax.conv_general_dilated(img, p['conv_w'], (1, 1), 'SAME',
                                       dimension_numbers=('NCHW', 'OIHW', 'NCHW'))
    dec = _relu(dec + p['conv_b'][None, :, None, None])
    roi = _roi_align(dec, imgpos, ROI)
    h = roi.reshape(roi.shape[0], -1)
    h = _relu(h @ p['emb1_w'] + p['emb1_b'])
    h = _relu(h @ p['emb2_w'] + p['emb2_b'])
    imgf = _relu(h @ p['emb3_w'] + p['emb3_b'])
    imgf = _dual_sc(imgf, src, dst, metaI, dinv, p, 'img', 1, p['w_img'][0])
    imgf = _dual_sc(imgf, src, dst, metaI, dinv, p, 'img', 2, p['w_img'][1])
    img_row = mlp2(imgf, 'post_img_row1', 'post_img_row2')
    img_col = mlp2(imgf, 'post_img_col1', 'post_img_col2')

    tf = mlp2(text, 'pret1', 'pret2')
    tf = _dual_sc(tf, src, dst, metaI, dinv, p, 'text', 1, p['w_text'][0])
    tf = _dual_sc(tf, src, dst, metaI, dinv, p, 'text', 2, p['w_text'][1])
    text_row = mlp2(tf, 'post_text_row1', 'post_text_row2')
    text_col = mlp2(tf, 'post_text_col1', 'post_text_col2')

    fr = mlp2(jnp.concatenate([box_row, img_row, text_row], axis=1), 'fus_row1', 'fus_row2')
    fc = mlp2(jnp.concatenate([box_col, img_col, text_col], axis=1), 'fus_col1', 'fus_col2')

    s, o = edge_pairing_index[0], edge_pairing_index[1]
    row_feat = jnp.concatenate([fr[s], fr[o]], axis=1)
    col_feat = jnp.concatenate([fc[s], fc[o]], axis=1)
    cls_row = _mlp2_pallas(row_feat, p['cls_row1_w'], p['cls_row1_b'],
                           p['cls_row2_w'], p['cls_row2_b'])
    cls_col = _mlp2_pallas(col_feat, p['cls_col1_w'], p['cls_col1_b'],
                           p['cls_col2_w'], p['cls_col2_b'])
    return (cls_row, cls_col)
